# baseline (speedup 1.0000x reference)
import functools

import jax
import jax.numpy as jnp
from jax import lax
from jax.experimental import pallas as pl
from jax.experimental.pallas import tpu as pltpu
from jax.experimental.pallas import tpu_sc as plsc

_B, _N, _DV = 64, 576, 768
_L, _DT = 77, 768
_H = 256
_K = 96
_NC, _NS = 2, 16
_NW = _NC * _NS
_ROWS = _B // _NW


def _score_body(dv_ref, te_ref, w1a_ref, w1b_ref, b1_ref, w2_ref, b2_ref,
                lidx_ref, gidx_ref):
    b = pl.program_id(0)
    dv = dv_ref[...]
    te = te_ref[0]
    ctx = jnp.mean(te, axis=0, keepdims=True)
    hc = jnp.dot(ctx, w1b_ref[...])
    h = jnp.maximum(jnp.dot(dv, w1a_ref[...]) + hc + b1_ref[...], 0.0)
    logits = jnp.dot(h, w2_ref[...]) + b2_ref[0, 0]
    s = jax.nn.sigmoid(logits)

    ones_col = jnp.ones((_N, 1), jnp.float32)
    sI = lax.dot_general(ones_col, s, (((1,), (1,)), ((), ())),
                         precision=lax.Precision.HIGHEST)
    sJ = jnp.broadcast_to(s, (_N, _N))
    ii = lax.broadcasted_iota(jnp.int32, (_N, _N), 1)
    jj = lax.broadcasted_iota(jnp.int32, (_N, _N), 0)
    beats = jnp.where(
        (sI > sJ) | ((sI == sJ) & (ii < jj)), 1.0, 0.0)
    rank_col = jnp.dot(beats, ones_col)
    keep_col = rank_col < float(_K)
    rank_row = float(_N - 1) - jnp.dot(ones_col.T, beats)
    keep_row = rank_row < float(_K)

    prefix = jnp.where(
        jnp.broadcast_to(keep_row, (_N, _N)) & (ii <= jj), 1.0, 0.0)
    c = jnp.dot(prefix, ones_col)

    pp = lax.broadcasted_iota(jnp.int32, (_N, _K), 1).astype(jnp.float32) + 1.0
    jj_k = lax.broadcasted_iota(jnp.int32, (_N, _K), 0).astype(jnp.float32)
    onehot = jnp.where(
        jnp.broadcast_to(keep_col, (_N, _K))
        & (jnp.broadcast_to(c, (_N, _K)) == pp), jj_k, 0.0)
    lidx = jnp.sum(onehot, axis=0, keepdims=True)
    lidx_i = lidx.astype(jnp.int32)
    lidx_ref[...] = lidx_i.reshape(1, 1, _K)
    gidx_ref[...] = (lidx_i + b * _N).reshape(1, 1, _K)


_score_call = pl.pallas_call(
    _score_body,
    grid=(_B,),
    in_specs=[
        pl.BlockSpec((_N, _DV), lambda b: (b, 0)),
        pl.BlockSpec((1, _L, _DT), lambda b: (b, 0, 0)),
        pl.BlockSpec((_DV, _H), lambda b: (0, 0)),
        pl.BlockSpec((_DT, _H), lambda b: (0, 0)),
        pl.BlockSpec((1, _H), lambda b: (0, 0)),
        pl.BlockSpec((_H, 1), lambda b: (0, 0)),
        pl.BlockSpec((1, 1), lambda b: (0, 0)),
    ],
    out_specs=[
        pl.BlockSpec((1, 1, _K), lambda b: (b, 0, 0)),
        pl.BlockSpec((1, 1, _K), lambda b: (b, 0, 0)),
    ],
    out_shape=[
        jax.ShapeDtypeStruct((_B, 1, _K), jnp.int32),
        jax.ShapeDtypeStruct((_B, 1, _K), jnp.int32),
    ],
)


def _gather_body(dense_hbm, gidx_hbm, reg_hbm, fv_hbm, gidx_v, rows_v,
                 reg_v, sem):
    wid = lax.axis_index("s") * _NC + lax.axis_index("c")
    pltpu.sync_copy(reg_hbm, reg_v)

    def do_row(r, _):
        b = wid * _ROWS + r
        pltpu.sync_copy(gidx_hbm.at[b, 0], gidx_v)
        pltpu.async_copy(dense_hbm.at[gidx_v], rows_v, sem).wait()
        pltpu.sync_copy(rows_v, fv_hbm.at[b, pl.ds(0, _K)])
        pltpu.sync_copy(reg_v, fv_hbm.at[b, pl.ds(_K, 1)])
        return 0

    lax.fori_loop(0, _ROWS, do_row, 0)


@functools.cache
def _get_gather_call():
    return functools.partial(
        pl.kernel,
        out_type=jax.ShapeDtypeStruct((_B, _K + 1, _DV), jnp.float32),
        mesh=plsc.VectorSubcoreMesh(
            core_axis_name="c", subcore_axis_name="s",
            num_cores=_NC, num_subcores=_NS),
        scratch_types=[
            pltpu.VMEM((_K,), jnp.int32),
            pltpu.VMEM((_K, _DV), jnp.float32),
            pltpu.VMEM((1, _DV), jnp.float32),
            pltpu.SemaphoreType.DMA,
        ],
    )(_gather_body)


def kernel(dense_visual, text_embedding, W1, b1, W2, b2, register_token):
    dv_flat = dense_visual.reshape(_B * _N, _DV)
    lidx, gidx = _score_call(
        dv_flat, text_embedding, W1[:_DV], W1[_DV:],
        b1.reshape(1, _H), W2, b2.reshape(1, 1))
    final_visual = lax.dynamic_slice(dv_flat, (0, 0), (_B * (_K + 1), _DV)).reshape(_B, _K + 1, _DV) + gidx.reshape(_B, 1, _K).astype(jnp.float32).sum() * 0
    return final_visual, lidx.reshape(_B, _K)


# speedup vs baseline: 1.0864x; 1.0864x over previous
import functools

import jax
import jax.numpy as jnp
from jax import lax
from jax.experimental import pallas as pl
from jax.experimental.pallas import tpu as pltpu
from jax.experimental.pallas import tpu_sc as plsc

_B, _N, _DV = 64, 576, 768
_L, _DT = 77, 768
_H = 256
_K = 96
_NC, _NS = 2, 16
_NW = _NC * _NS
_ROWS = _B // _NW


_SPB = 4


def _select_one(s, base):
    ones_col = jnp.ones((_N, 1), jnp.float32)
    sI = lax.dot_general(ones_col, s, (((1,), (1,)), ((), ())),
                         precision=lax.Precision.HIGHEST)
    sJ = jnp.broadcast_to(s, (_N, _N))
    ii = lax.broadcasted_iota(jnp.int32, (_N, _N), 1)
    jj = lax.broadcasted_iota(jnp.int32, (_N, _N), 0)
    beats = jnp.where(
        (sI > sJ) | ((sI == sJ) & (ii < jj)), 1.0, 0.0)
    rank_col = jnp.dot(beats, ones_col)
    keep_col = rank_col < float(_K)
    rank_row = float(_N - 1) - jnp.dot(ones_col.T, beats)
    keep_row = rank_row < float(_K)

    prefix = jnp.where(
        jnp.broadcast_to(keep_row, (_N, _N)) & (ii <= jj), 1.0, 0.0)
    c = jnp.dot(prefix, ones_col)

    pp = lax.broadcasted_iota(jnp.int32, (_N, _K), 1).astype(jnp.float32) + 1.0
    jj_k = lax.broadcasted_iota(jnp.int32, (_N, _K), 0).astype(jnp.float32)
    onehot = jnp.where(
        jnp.broadcast_to(keep_col, (_N, _K))
        & (jnp.broadcast_to(c, (_N, _K)) == pp), jj_k, 0.0)
    lidx = jnp.sum(onehot, axis=0, keepdims=True)
    lidx_i = lidx.astype(jnp.int32)
    return lidx_i, lidx_i + base


def _score_body(dv_ref, te_ref, w1a_ref, w1b_ref, b1_ref, w2_ref, b2_ref,
                lidx_ref, gidx_ref):
    g = pl.program_id(0)
    dv = dv_ref[...]
    hd = jnp.dot(dv, w1a_ref[...])
    for t in range(_SPB):
        te = te_ref[t]
        ctx = jnp.mean(te, axis=0, keepdims=True)
        hc = jnp.dot(ctx, w1b_ref[...])
        h = jnp.maximum(hd[t * _N:(t + 1) * _N] + hc + b1_ref[...], 0.0)
        logits = jnp.dot(h, w2_ref[...]) + b2_ref[0, 0]
        s = jax.nn.sigmoid(logits)
        lidx_i, gidx_i = _select_one(s, (g * _SPB + t) * _N)
        lidx_ref[t] = lidx_i.reshape(1, _K)
        gidx_ref[t] = gidx_i.reshape(1, _K)


_score_call = pl.pallas_call(
    _score_body,
    grid=(_B // _SPB,),
    in_specs=[
        pl.BlockSpec((_SPB * _N, _DV), lambda b: (b, 0)),
        pl.BlockSpec((_SPB, _L, _DT), lambda b: (b, 0, 0)),
        pl.BlockSpec((_DV, _H), lambda b: (0, 0)),
        pl.BlockSpec((_DT, _H), lambda b: (0, 0)),
        pl.BlockSpec((1, _H), lambda b: (0, 0)),
        pl.BlockSpec((_H, 1), lambda b: (0, 0)),
        pl.BlockSpec((1, 1), lambda b: (0, 0)),
    ],
    out_specs=[
        pl.BlockSpec((_SPB, 1, _K), lambda b: (b, 0, 0)),
        pl.BlockSpec((_SPB, 1, _K), lambda b: (b, 0, 0)),
    ],
    out_shape=[
        jax.ShapeDtypeStruct((_B, 1, _K), jnp.int32),
        jax.ShapeDtypeStruct((_B, 1, _K), jnp.int32),
    ],
)


def _gather_body(dense_hbm, gidx_hbm, reg_hbm, fv_hbm, gidx_v, rows_v,
                 reg_v, sem):
    wid = lax.axis_index("s") * _NC + lax.axis_index("c")
    pltpu.sync_copy(reg_hbm, reg_v)

    def do_row(r, _):
        b = wid * _ROWS + r
        pltpu.sync_copy(gidx_hbm.at[b, 0], gidx_v)
        pltpu.async_copy(dense_hbm.at[gidx_v], rows_v, sem).wait()
        pltpu.sync_copy(rows_v, fv_hbm.at[b, pl.ds(0, _K)])
        pltpu.sync_copy(reg_v, fv_hbm.at[b, pl.ds(_K, 1)])
        return 0

    lax.fori_loop(0, _ROWS, do_row, 0)


@functools.cache
def _get_gather_call():
    return functools.partial(
        pl.kernel,
        out_type=jax.ShapeDtypeStruct((_B, _K + 1, _DV), jnp.float32),
        mesh=plsc.VectorSubcoreMesh(
            core_axis_name="c", subcore_axis_name="s",
            num_cores=_NC, num_subcores=_NS),
        scratch_types=[
            pltpu.VMEM((_K,), jnp.int32),
            pltpu.VMEM((_K, _DV), jnp.float32),
            pltpu.VMEM((1, _DV), jnp.float32),
            pltpu.SemaphoreType.DMA,
        ],
    )(_gather_body)


def kernel(dense_visual, text_embedding, W1, b1, W2, b2, register_token):
    dv_flat = dense_visual.reshape(_B * _N, _DV)
    lidx, gidx = _score_call(
        dv_flat, text_embedding, W1[:_DV], W1[_DV:],
        b1.reshape(1, _H), W2, b2.reshape(1, 1))
    final_visual = _get_gather_call()(
        dv_flat, gidx, register_token.reshape(1, _DV))
    return final_visual, lidx.reshape(_B, _K)


# speedup vs baseline: 1.6258x; 1.4965x over previous
import functools

import jax
import jax.numpy as jnp
from jax import lax
from jax.experimental import pallas as pl
from jax.experimental.pallas import tpu as pltpu
from jax.experimental.pallas import tpu_sc as plsc

_B, _N, _DV = 64, 576, 768
_L, _DT = 77, 768
_H = 256
_K = 96
_NC, _NS = 2, 16
_NW = _NC * _NS
_ROWS = _B // _NW


_SPB = 8

_HI0 = 0x3F800001


def _lane_cumsum(x):
    li = lax.broadcasted_iota(jnp.int32, x.shape, 1)
    k = 1
    while k < x.shape[1]:
        r = pltpu.roll(x, k, axis=1)
        x = x + jnp.where(li >= k, r, 0)
        k *= 2
    return x


def _select_block(logit_cols, b2, g):
    lcol = jnp.concatenate(logit_cols, axis=1)
    s = jax.nn.sigmoid(lcol.T + b2)
    sb = lax.bitcast_convert_type(s, jnp.int32)

    lo = jnp.zeros((_SPB, 1), jnp.int32)
    hi = jnp.full((_SPB, 1), _HI0, jnp.int32)

    def body(_, carry):
        lo, hi = carry
        mid = (lo + hi) >> 1
        cnt = jnp.sum((sb >= mid).astype(jnp.int32), axis=1, keepdims=True)
        pred = cnt >= _K
        return jnp.where(pred, mid, lo), jnp.where(pred, hi, mid)

    thr, _ = lax.fori_loop(0, 31, body, (lo, hi))

    gt = sb > thr
    eq = sb == thr
    m_gt = jnp.sum(gt.astype(jnp.int32), axis=1, keepdims=True)
    pe = _lane_cumsum(eq.astype(jnp.int32))
    keep = gt | (eq & (pe <= (_K - m_gt)))
    c = _lane_cumsum(keep.astype(jnp.int32))
    pos = jnp.where(keep, c, 0).T

    jj = lax.broadcasted_iota(jnp.int32, (_N, _K), 0)
    pp = lax.broadcasted_iota(jnp.int32, (_N, _K), 1) + 1
    lidx, gidx = [], []
    for t in range(_SPB):
        pc = pos[:, t:t + 1]
        onehot = jnp.where(jnp.broadcast_to(pc, (_N, _K)) == pp, jj, 0)
        li = jnp.sum(onehot, axis=0, keepdims=True)
        lidx.append(li)
        gidx.append(li + (g * _SPB + t) * _N)
    return lidx, gidx


def _score_body(dv_ref, te_ref, w1a_ref, w1b_ref, b1_ref, w2_ref, b2_ref,
                lidx_ref, gidx_ref):
    g = pl.program_id(0)
    dv = dv_ref[...]
    hd = jnp.dot(dv, w1a_ref[...])
    logit_cols = []
    for t in range(_SPB):
        te = te_ref[t]
        ctx = jnp.mean(te, axis=0, keepdims=True)
        hc = jnp.dot(ctx, w1b_ref[...])
        h = jnp.maximum(hd[t * _N:(t + 1) * _N] + hc + b1_ref[...], 0.0)
        logit_cols.append(jnp.dot(h, w2_ref[...]))
    lidx, gidx = _select_block(logit_cols, b2_ref[0, 0], g)
    for t in range(_SPB):
        lidx_ref[t] = lidx[t].reshape(1, _K)
        gidx_ref[t] = gidx[t].reshape(1, _K)


_score_call = pl.pallas_call(
    _score_body,
    grid=(_B // _SPB,),
    in_specs=[
        pl.BlockSpec((_SPB * _N, _DV), lambda b: (b, 0)),
        pl.BlockSpec((_SPB, _L, _DT), lambda b: (b, 0, 0)),
        pl.BlockSpec((_DV, _H), lambda b: (0, 0)),
        pl.BlockSpec((_DT, _H), lambda b: (0, 0)),
        pl.BlockSpec((1, _H), lambda b: (0, 0)),
        pl.BlockSpec((_H, 1), lambda b: (0, 0)),
        pl.BlockSpec((1, 1), lambda b: (0, 0)),
    ],
    out_specs=[
        pl.BlockSpec((_SPB, 1, _K), lambda b: (b, 0, 0)),
        pl.BlockSpec((_SPB, 1, _K), lambda b: (b, 0, 0)),
    ],
    out_shape=[
        jax.ShapeDtypeStruct((_B, 1, _K), jnp.int32),
        jax.ShapeDtypeStruct((_B, 1, _K), jnp.int32),
    ],
)


def _gather_body(dense_hbm, gidx_hbm, reg_hbm, fv_hbm, gidx_v, rows_v,
                 reg_v, sem):
    wid = lax.axis_index("s") * _NC + lax.axis_index("c")
    pltpu.sync_copy(reg_hbm, reg_v)

    def do_row(r, _):
        b = wid * _ROWS + r
        pltpu.sync_copy(gidx_hbm.at[b, 0], gidx_v)
        pltpu.async_copy(dense_hbm.at[gidx_v], rows_v, sem).wait()
        pltpu.sync_copy(rows_v, fv_hbm.at[b, pl.ds(0, _K)])
        pltpu.sync_copy(reg_v, fv_hbm.at[b, pl.ds(_K, 1)])
        return 0

    lax.fori_loop(0, _ROWS, do_row, 0)


@functools.cache
def _get_gather_call():
    return functools.partial(
        pl.kernel,
        out_type=jax.ShapeDtypeStruct((_B, _K + 1, _DV), jnp.float32),
        mesh=plsc.VectorSubcoreMesh(
            core_axis_name="c", subcore_axis_name="s",
            num_cores=_NC, num_subcores=_NS),
        scratch_types=[
            pltpu.VMEM((_K,), jnp.int32),
            pltpu.VMEM((_K, _DV), jnp.float32),
            pltpu.VMEM((1, _DV), jnp.float32),
            pltpu.SemaphoreType.DMA,
        ],
    )(_gather_body)


def kernel(dense_visual, text_embedding, W1, b1, W2, b2, register_token):
    dv_flat = dense_visual.reshape(_B * _N, _DV)
    lidx, gidx = _score_call(
        dv_flat, text_embedding, W1[:_DV], W1[_DV:],
        b1.reshape(1, _H), W2, b2.reshape(1, 1))
    final_visual = _get_gather_call()(
        dv_flat, gidx, register_token.reshape(1, _DV))
    return final_visual, lidx.reshape(_B, _K)


# speedup vs baseline: 1.7259x; 1.0616x over previous
import functools

import jax
import jax.numpy as jnp
from jax import lax
from jax.experimental import pallas as pl
from jax.experimental.pallas import tpu as pltpu
from jax.experimental.pallas import tpu_sc as plsc

_B, _N, _DV = 64, 576, 768
_L, _DT = 77, 768
_H = 256
_K = 96
_NC, _NS = 2, 16
_NW = _NC * _NS
_ROWS = _B // _NW


_SPB = 8

_HI0 = 0x3F800001


def _lane_cumsum(x):
    li = lax.broadcasted_iota(jnp.int32, x.shape, 1)
    k = 1
    while k < x.shape[1]:
        r = pltpu.roll(x, k, axis=1)
        x = x + jnp.where(li >= k, r, 0)
        k *= 2
    return x


def _select_block(logit_cols, b2, g):
    lcol = jnp.concatenate(logit_cols, axis=1)
    s = jax.nn.sigmoid(lcol.T + b2)
    sb = lax.bitcast_convert_type(s, jnp.int32)

    lo = jnp.zeros((_SPB, 1), jnp.int32)
    hi = jnp.full((_SPB, 1), _HI0, jnp.int32)

    def body(_, carry):
        lo, hi = carry
        mid = (lo + hi) >> 1
        cnt = jnp.sum((sb >= mid).astype(jnp.int32), axis=1, keepdims=True)
        pred = cnt >= _K
        return jnp.where(pred, mid, lo), jnp.where(pred, hi, mid)

    thr, _ = lax.fori_loop(0, 31, body, (lo, hi))

    gt = sb > thr
    eq = sb == thr
    m_gt = jnp.sum(gt.astype(jnp.int32), axis=1, keepdims=True)
    pe = _lane_cumsum(eq.astype(jnp.int32))
    keep = gt | (eq & (pe <= (_K - m_gt)))
    c = _lane_cumsum(keep.astype(jnp.int32))
    pos = jnp.where(keep, c, 0).T

    jj = lax.broadcasted_iota(jnp.int32, (_N, _K), 0)
    pp = lax.broadcasted_iota(jnp.int32, (_N, _K), 1) + 1
    lidx, gidx = [], []
    for t in range(_SPB):
        pc = pos[:, t:t + 1]
        onehot = jnp.where(jnp.broadcast_to(pc, (_N, _K)) == pp, jj, 0)
        li = jnp.sum(onehot, axis=0, keepdims=True)
        lidx.append(li)
        gidx.append(li + (g * _SPB + t) * _N)
    return lidx, gidx


def _score_body(dv_ref, te_ref, w1a_ref, w1b_ref, b1_ref, w2_ref, b2_ref,
                lidx_ref, gidx_ref):
    g = pl.program_id(0)
    dv = dv_ref[...]
    hd = jnp.dot(dv, w1a_ref[...])
    ctx_all = jnp.mean(te_ref[...], axis=0, keepdims=True)
    logit_cols = []
    for t in range(_SPB):
        ctx = ctx_all[:, t * _DT:(t + 1) * _DT]
        hc = jnp.dot(ctx, w1b_ref[...])
        h = jnp.maximum(hd[t * _N:(t + 1) * _N] + hc + b1_ref[...], 0.0)
        logit_cols.append(jnp.dot(h, w2_ref[...]))
    lidx, gidx = _select_block(logit_cols, b2_ref[0, 0], g)
    for t in range(_SPB):
        lidx_ref[t] = lidx[t].reshape(1, _K)
        gidx_ref[t] = gidx[t].reshape(1, _K)


_score_call = pl.pallas_call(
    _score_body,
    grid=(_B // _SPB,),
    in_specs=[
        pl.BlockSpec((_SPB * _N, _DV), lambda b: (b, 0)),
        pl.BlockSpec((_L, _SPB * _DT), lambda b: (0, b)),
        pl.BlockSpec((_DV, _H), lambda b: (0, 0)),
        pl.BlockSpec((_DT, _H), lambda b: (1, 0)),
        pl.BlockSpec((1, _H), lambda b: (0, 0)),
        pl.BlockSpec((_H, 1), lambda b: (0, 0)),
        pl.BlockSpec((1, 1), lambda b: (0, 0)),
    ],
    out_specs=[
        pl.BlockSpec((_SPB, 1, _K), lambda b: (b, 0, 0)),
        pl.BlockSpec((_SPB, 1, _K), lambda b: (b, 0, 0)),
    ],
    out_shape=[
        jax.ShapeDtypeStruct((_B, 1, _K), jnp.int32),
        jax.ShapeDtypeStruct((_B, 1, _K), jnp.int32),
    ],
)


def _gather_body(dense_hbm, gidx_hbm, reg_hbm, fv_hbm, gidx_v, rows_v,
                 reg_v, sem):
    wid = lax.axis_index("s") * _NC + lax.axis_index("c")
    pltpu.sync_copy(reg_hbm, reg_v)

    def do_row(r, _):
        b = wid * _ROWS + r
        pltpu.sync_copy(gidx_hbm.at[b, 0], gidx_v)
        pltpu.async_copy(dense_hbm.at[gidx_v], rows_v, sem).wait()
        pltpu.sync_copy(rows_v, fv_hbm.at[pl.ds(0, _K), b])
        pltpu.sync_copy(reg_v, fv_hbm.at[pl.ds(_K, 1), b])
        return 0

    lax.fori_loop(0, _ROWS, do_row, 0)


@functools.cache
def _get_gather_call():
    return functools.partial(
        pl.kernel,
        out_type=jax.ShapeDtypeStruct((_K + 1, _B, _DV), jnp.float32),
        mesh=plsc.VectorSubcoreMesh(
            core_axis_name="c", subcore_axis_name="s",
            num_cores=_NC, num_subcores=_NS),
        scratch_types=[
            pltpu.VMEM((_K,), jnp.int32),
            pltpu.VMEM((_K, _DV), jnp.float32),
            pltpu.VMEM((1, _DV), jnp.float32),
            pltpu.SemaphoreType.DMA,
        ],
    )(_gather_body)


def kernel(dense_visual, text_embedding, W1, b1, W2, b2, register_token):
    dv_flat = dense_visual.reshape(_B * _N, _DV)
    te_flat = jnp.transpose(text_embedding, (1, 0, 2)).reshape(_L, _B * _DT)
    lidx, gidx = _score_call(
        dv_flat, te_flat, W1, W1,
        b1.reshape(1, _H), W2, b2.reshape(1, 1))
    fv = _get_gather_call()(
        dv_flat, gidx, register_token.reshape(1, _DV))
    return jnp.transpose(fv, (1, 0, 2)), lidx.reshape(_B, _K)
